# Initial kernel scaffold; baseline (speedup 1.0000x reference)
import jax
import jax.numpy as jnp
from jax.experimental import pallas as pl


def kernel(xyz, points, W1, b1, W2, b2, W3, b3):
    raise NotImplementedError("write your pallas kernel here")


# speedup vs baseline: 7.5553x; 7.5553x over previous
import functools

import jax
import jax.numpy as jnp
from jax import lax
from jax.experimental import pallas as pl
from jax.experimental.pallas import tpu as pltpu
from jax.experimental.pallas import tpu_sc as plsc

NB = 4
NP = 8192
NC = 1024
NS = 32
RAD2 = 0.4 * 0.4
SBLK = 128
DPAD = 128


def _fps_body(xyz_ref, nxyz_ref, dist_ref, acc_ref):
    lane = lax.broadcasted_iota(jnp.int32, (1, NP), 1)
    siota = lax.broadcasted_iota(jnp.int32, (1, NC), 1)
    dist_ref[...] = jnp.full((1, NP), 1e10, jnp.float32)
    acc_ref[...] = jnp.zeros((8, NC), jnp.float32)
    xyz = xyz_ref[...]

    def body(i, far):
        oh = (lane == far).astype(jnp.float32)
        c = jnp.sum(xyz * oh, axis=1, keepdims=True)
        d = jnp.sum((xyz - c) ** 2, axis=0, keepdims=True)
        dist = jnp.minimum(dist_ref[...], d)
        dist_ref[...] = dist
        sel = siota == i
        acc_ref[...] = jnp.where(sel, c, acc_ref[...])
        m = jnp.max(dist)
        far_new = jnp.min(jnp.where(dist == m, lane, NP))
        return far_new

    lax.fori_loop(0, NC, body, jnp.int32(0))
    nxyz_ref[...] = acc_ref[...]


def _run_fps(xyz_pad):
    return pl.pallas_call(
        _fps_body,
        grid=(NB,),
        in_specs=[pl.BlockSpec((8, NP), lambda b: (b, 0))],
        out_specs=pl.BlockSpec((8, NC), lambda b: (b, 0)),
        out_shape=jax.ShapeDtypeStruct((NB * 8, NC), jnp.float32),
        scratch_shapes=[
            pltpu.VMEM((1, NP), jnp.float32),
            pltpu.VMEM((8, NC), jnp.float32),
        ],
    )(xyz_pad)


def _bq_body(xyz_ref, nx_ref, idx_ref):
    t = pl.program_id(0)
    b = t // (NC // SBLK)
    pts = xyz_ref[...]
    cen = nx_ref[...]
    s2 = jnp.sum(pts * pts, axis=0, keepdims=True)
    s1 = jnp.sum(cen * cen, axis=1, keepdims=True)
    cross = jnp.dot(cen, pts, preferred_element_type=jnp.float32)
    d2 = s1 + s2 - 2.0 * cross
    lane = lax.broadcasted_iota(jnp.int32, (1, NP), 1)
    masked = jnp.where(d2 <= RAD2, lane, NP)
    kiota = lax.broadcasted_iota(jnp.int32, (1, NS), 1)
    out = jnp.zeros((SBLK, NS), jnp.int32)
    first = None
    for k in range(NS):
        m = jnp.min(masked, axis=1, keepdims=True)
        if k == 0:
            first = m
            col = m
        else:
            col = jnp.where(m == NP, first, m)
        out = jnp.where(kiota == k, col, out)
        if k < NS - 1:
            masked = jnp.where(masked == m, NP, masked)
    idx_ref[...] = out + b * NP


def _run_bq(xyz_pad, nxyz_rows):
    return pl.pallas_call(
        _bq_body,
        grid=(NB * NC // SBLK,),
        in_specs=[
            pl.BlockSpec((8, NP), lambda t: (t // (NC // SBLK), 0)),
            pl.BlockSpec((SBLK, 8), lambda t: (t, 0)),
        ],
        out_specs=pl.BlockSpec((SBLK, NS), lambda t: (t, 0)),
        out_shape=jax.ShapeDtypeStruct((NB * NC, NS), jnp.int32),
    )(xyz_pad, nxyz_rows)


def _sc_gather(table, idx):
    total = idx.shape[0]
    info = plsc.get_sparse_core_info()
    nworkers = info.num_cores * info.num_subcores
    per_w = total // nworkers
    ch = 512
    nch = per_w // ch
    mesh = plsc.VectorSubcoreMesh(core_axis_name="c", subcore_axis_name="s")

    @functools.partial(
        pl.kernel,
        mesh=mesh,
        out_type=jax.ShapeDtypeStruct((total, DPAD), jnp.float32),
        scratch_types=[
            pltpu.VMEM((ch,), jnp.int32),
            pltpu.VMEM((ch, DPAD), jnp.float32),
            pltpu.SemaphoreType.DMA,
        ],
    )
    def gather_k(table_hbm, idx_hbm, out_hbm, idx_v, rows_v, sem):
        wid = lax.axis_index("s") * info.num_cores + lax.axis_index("c")
        base = wid * per_w

        def chunk(j, carry):
            off = base + j * ch
            pltpu.sync_copy(idx_hbm.at[pl.ds(off, ch)], idx_v)
            pltpu.async_copy(table_hbm.at[idx_v], rows_v, sem).wait()
            pltpu.sync_copy(rows_v, out_hbm.at[pl.ds(off, ch)])
            return carry

        lax.fori_loop(0, nch, chunk, 0)

    return gather_k(table, idx)


def _mlp_body(g_ref, nx_ref, w1_ref, b1_ref, w2_ref, b2_ref, w3_ref,
              b3_ref, w1x_ref, out_ref):
    g = g_ref[...]
    h = jnp.dot(g, w1_ref[...], preferred_element_type=jnp.float32)
    h = h + b1_ref[...]
    corr = jnp.dot(nx_ref[...], w1x_ref[...],
                   preferred_element_type=jnp.float32)
    h = h.reshape(SBLK, NS, 128) - corr[:, None, :]
    h = jnp.maximum(h, 0.0).reshape(SBLK * NS, 128)
    h = jnp.dot(h, w2_ref[...], preferred_element_type=jnp.float32)
    h = jnp.maximum(h + b2_ref[...], 0.0)
    h = jnp.dot(h, w3_ref[...], preferred_element_type=jnp.float32)
    h = jnp.maximum(h + b3_ref[...], 0.0)
    out_ref[...] = jnp.max(h.reshape(SBLK, NS, 256), axis=1)


def _run_mlp(gathered, nxyz_rows, w1p, b1r, w2, b2r, w3, b3r, w1x):
    return pl.pallas_call(
        _mlp_body,
        grid=(NB * NC // SBLK,),
        in_specs=[
            pl.BlockSpec((SBLK * NS, DPAD), lambda t: (t, 0)),
            pl.BlockSpec((SBLK, 8), lambda t: (t, 0)),
            pl.BlockSpec((DPAD, 128), lambda t: (0, 0)),
            pl.BlockSpec((1, 128), lambda t: (0, 0)),
            pl.BlockSpec((128, 128), lambda t: (0, 0)),
            pl.BlockSpec((1, 128), lambda t: (0, 0)),
            pl.BlockSpec((128, 256), lambda t: (0, 0)),
            pl.BlockSpec((1, 256), lambda t: (0, 0)),
            pl.BlockSpec((8, 128), lambda t: (0, 0)),
        ],
        out_specs=pl.BlockSpec((SBLK, 256), lambda t: (t, 0)),
        out_shape=jax.ShapeDtypeStruct((NB * NC, 256), jnp.float32),
    )(gathered, nxyz_rows, w1p, b1r, w2, b2r, w3, b3r, w1x)


def kernel(xyz, points, W1, b1, W2, b2, W3, b3):
    feats = points[:, 3:, :]
    zpad = jnp.zeros((NB, 5, NP), jnp.float32)
    xyz_pad = jnp.concatenate([xyz, zpad], axis=1).reshape(NB * 8, NP)

    nxyz8 = _run_fps(xyz_pad)
    nxyz_rows = (nxyz8.reshape(NB, 8, NC)
                 .transpose(0, 2, 1).reshape(NB * NC, 8))

    idx = _run_bq(xyz_pad, nxyz_rows)

    table = jnp.concatenate(
        [jnp.transpose(xyz, (0, 2, 1)),
         jnp.transpose(feats, (0, 2, 1)),
         jnp.zeros((NB, NP, DPAD - 67), jnp.float32)],
        axis=2).reshape(NB * NP, DPAD)
    gathered = _sc_gather(table, idx.reshape(-1))

    w1p = jnp.concatenate(
        [W1, jnp.zeros((DPAD - 67, 128), jnp.float32)], axis=0)
    w1x = jnp.concatenate(
        [W1[:3], jnp.zeros((5, 128), jnp.float32)], axis=0)
    nf = _run_mlp(gathered, nxyz_rows, w1p, b1.reshape(1, 128),
                  W2, b2.reshape(1, 128), W3, b3.reshape(1, 256), w1x)

    new_xyz = nxyz_rows.reshape(NB, NC, 8)[:, :, :3].transpose(0, 2, 1)
    return new_xyz, nf.reshape(NB, NC, 256).transpose(0, 2, 1)
